# initial kernel scaffold (unmeasured)
import jax
import jax.numpy as jnp
from jax import lax
from jax.experimental import pallas as pl
from jax.experimental.pallas import tpu as pltpu

N_DEV = 16
M = 4096
N = 2048
CHUNK = M // N_DEV


def kernel(x, w_mat):
    def body(x_ref, w_ref, out_ref, comm_ref, send_sems, recv_sems):
        my = lax.axis_index("i")
        left = lax.rem(my + N_DEV - 1, N_DEV)
        right = lax.rem(my + 1, N_DEV)

        barrier_sem = pltpu.get_barrier_semaphore()
        for nbr in (left, right):
            pl.semaphore_signal(
                barrier_sem, inc=1,
                device_id=(nbr,), device_id_type=pl.DeviceIdType.MESH,
            )
        pl.semaphore_wait(barrier_sem, 2)

        xb = x_ref[...].astype(jnp.bfloat16)
        wb = w_ref[...].astype(jnp.bfloat16)
        out_ref[...] = jnp.dot(xb, wb, preferred_element_type=jnp.float32)

        def rows(c):
            return pl.ds(c * CHUNK, CHUNK)

        comm_ref[0, :, :] = out_ref[rows(my), :]

        for h in range(2 * (N_DEV - 1)):
            send_slot = h % 2
            recv_slot = (h + 1) % 2
            rdma = pltpu.make_async_remote_copy(
                src_ref=comm_ref.at[send_slot],
                dst_ref=comm_ref.at[recv_slot],
                send_sem=send_sems.at[send_slot],
                recv_sem=recv_sems.at[recv_slot],
                device_id=(right,),
                device_id_type=pl.DeviceIdType.MESH,
            )
            rdma.start()
            rdma.wait()

            if h < N_DEV - 1:
                c = lax.rem(my + N_DEV - h - 1, N_DEV)
                summed = comm_ref[recv_slot, :, :] + out_ref[rows(c), :]
                comm_ref[recv_slot, :, :] = summed
                if h == N_DEV - 2:
                    out_ref[rows(c), :] = summed
            else:
                t = h - (N_DEV - 1)
                c = lax.rem(my + N_DEV - t, N_DEV)
                out_ref[rows(c), :] = comm_ref[recv_slot, :, :]

    return pl.pallas_call(
        body,
        out_shape=jax.ShapeDtypeStruct((M, N), jnp.float32),
        in_specs=[
            pl.BlockSpec(memory_space=pltpu.VMEM),
            pl.BlockSpec(memory_space=pltpu.VMEM),
        ],
        out_specs=pl.BlockSpec(memory_space=pltpu.VMEM),
        scratch_shapes=[
            pltpu.VMEM((2, CHUNK, N), jnp.float32),
            pltpu.SemaphoreType.DMA((2,)),
            pltpu.SemaphoreType.DMA((2,)),
        ],
        compiler_params=pltpu.CompilerParams(collective_id=0),
    )(x, w_mat)


# baseline (device time: 779789 ns/iter reference)
import jax
import jax.numpy as jnp
from jax import lax
from jax.experimental import pallas as pl
from jax.experimental.pallas import tpu as pltpu

N_DEV = 16
M = 4096
N = 2048
CHUNK = M // N_DEV


def kernel(x, w_mat):
    def body(x_ref, w_ref, out_ref, comm_ref, send_sems, recv_sems):
        my = lax.axis_index("i")
        left = lax.rem(my + N_DEV - 1, N_DEV)
        right = lax.rem(my + 1, N_DEV)

        barrier_sem = pltpu.get_barrier_semaphore()
        for nbr in (left, right):
            pl.semaphore_signal(
                barrier_sem, inc=1,
                device_id=(nbr,), device_id_type=pl.DeviceIdType.MESH,
            )
        pl.semaphore_wait(barrier_sem, 2)

        xb = x_ref[...].astype(jnp.bfloat16)
        wb = w_ref[...].astype(jnp.bfloat16)
        out_ref[...] = jnp.dot(xb, wb, preferred_element_type=jnp.float32)

        def rows(c):
            return pl.ds(c * CHUNK, CHUNK)

        comm_ref[0, :, :] = out_ref[rows(my), :]

        for h in range(2 * (N_DEV - 1)):
            send_slot = h % 2
            recv_slot = (h + 1) % 2
            rdma = pltpu.make_async_remote_copy(
                src_ref=comm_ref.at[send_slot],
                dst_ref=comm_ref.at[recv_slot],
                send_sem=send_sems.at[send_slot],
                recv_sem=recv_sems.at[recv_slot],
                device_id=(right,),
                device_id_type=pl.DeviceIdType.MESH,
            )
            rdma.start()
            rdma.wait()

            if h < N_DEV - 1:
                c = lax.rem(my + N_DEV - h - 1, N_DEV)
                summed = comm_ref[recv_slot, :, :] + out_ref[rows(c), :]
                comm_ref[recv_slot, :, :] = summed
                if h == N_DEV - 2:
                    out_ref[rows(c), :] = summed
            else:
                t = h - (N_DEV - 1)
                c = lax.rem(my + N_DEV - t, N_DEV)
                out_ref[rows(c), :] = comm_ref[recv_slot, :, :]

    return pl.pallas_call(
        body,
        out_shape=jax.ShapeDtypeStruct((M, N), jnp.float32),
        in_specs=[
            pl.BlockSpec(memory_space=pltpu.VMEM),
            pl.BlockSpec(memory_space=pltpu.VMEM),
        ],
        out_specs=pl.BlockSpec(memory_space=pltpu.VMEM),
        scratch_shapes=[
            pltpu.VMEM((2, CHUNK, N), jnp.float32),
            pltpu.SemaphoreType.DMA((2,)),
            pltpu.SemaphoreType.DMA((2,)),
        ],
        compiler_params=pltpu.CompilerParams(
            collective_id=0, vmem_limit_bytes=100 * 1024 * 1024
        ),
    )(x, w_mat)


# device time: 313971 ns/iter; 2.4836x vs baseline; 2.4836x over previous
import jax
import jax.numpy as jnp
from jax import lax
from jax.experimental import pallas as pl
from jax.experimental.pallas import tpu as pltpu

N_DEV = 16
M = 4096
N = 2048
CHUNK = M // N_DEV
HALF = N // 2


def kernel(x, w_mat):
    def body(x_ref, w_ref, out_ref, comm_r, comm_l,
             send_r, recv_r, send_l, recv_l):
        my = lax.axis_index("i")
        left = lax.rem(my + N_DEV - 1, N_DEV)
        right = lax.rem(my + 1, N_DEV)

        barrier_sem = pltpu.get_barrier_semaphore()
        for nbr in (left, right):
            pl.semaphore_signal(
                barrier_sem, inc=1,
                device_id=(nbr,), device_id_type=pl.DeviceIdType.MESH,
            )
        pl.semaphore_wait(barrier_sem, 2)

        xb = x_ref[...].astype(jnp.bfloat16)
        wb = w_ref[...].astype(jnp.bfloat16)
        out_ref[...] = jnp.dot(xb, wb, preferred_element_type=jnp.float32)

        def rows(c):
            return pl.ds(c * CHUNK, CHUNK)

        cols_r = pl.ds(0, HALF)
        cols_l = pl.ds(HALF, HALF)

        comm_r[0, :, :] = out_ref[rows(my), cols_r].astype(jnp.bfloat16)
        comm_l[0, :, :] = out_ref[rows(my), cols_l].astype(jnp.bfloat16)

        for h in range(2 * (N_DEV - 1)):
            s_slot = h % 2
            r_slot = (h + 1) % 2
            rdma_r = pltpu.make_async_remote_copy(
                src_ref=comm_r.at[s_slot],
                dst_ref=comm_r.at[r_slot],
                send_sem=send_r.at[s_slot],
                recv_sem=recv_r.at[r_slot],
                device_id=(right,),
                device_id_type=pl.DeviceIdType.MESH,
            )
            rdma_l = pltpu.make_async_remote_copy(
                src_ref=comm_l.at[s_slot],
                dst_ref=comm_l.at[r_slot],
                send_sem=send_l.at[s_slot],
                recv_sem=recv_l.at[r_slot],
                device_id=(left,),
                device_id_type=pl.DeviceIdType.MESH,
            )
            rdma_r.start()
            rdma_l.start()
            rdma_r.wait()
            rdma_l.wait()

            if h < N_DEV - 1:
                c = lax.rem(my + N_DEV - h - 1, N_DEV)
                summed = (comm_r[r_slot, :, :].astype(jnp.float32)
                          + out_ref[rows(c), cols_r])
                comm_r[r_slot, :, :] = summed.astype(jnp.bfloat16)
                if h == N_DEV - 2:
                    out_ref[rows(c), cols_r] = summed
                c = lax.rem(my + h + 1, N_DEV)
                summed = (comm_l[r_slot, :, :].astype(jnp.float32)
                          + out_ref[rows(c), cols_l])
                comm_l[r_slot, :, :] = summed.astype(jnp.bfloat16)
                if h == N_DEV - 2:
                    out_ref[rows(c), cols_l] = summed
            else:
                t = h - (N_DEV - 1)
                c = lax.rem(my + N_DEV - t, N_DEV)
                out_ref[rows(c), cols_r] = comm_r[r_slot, :, :].astype(
                    jnp.float32)
                c = lax.rem(my + t, N_DEV)
                out_ref[rows(c), cols_l] = comm_l[r_slot, :, :].astype(
                    jnp.float32)

    comm_shape = (2, CHUNK, HALF)
    return pl.pallas_call(
        body,
        out_shape=jax.ShapeDtypeStruct((M, N), jnp.float32),
        in_specs=[
            pl.BlockSpec(memory_space=pltpu.VMEM),
            pl.BlockSpec(memory_space=pltpu.VMEM),
        ],
        out_specs=pl.BlockSpec(memory_space=pltpu.VMEM),
        scratch_shapes=[
            pltpu.VMEM(comm_shape, jnp.bfloat16),
            pltpu.VMEM(comm_shape, jnp.bfloat16),
            pltpu.SemaphoreType.DMA((2,)),
            pltpu.SemaphoreType.DMA((2,)),
            pltpu.SemaphoreType.DMA((2,)),
            pltpu.SemaphoreType.DMA((2,)),
        ],
        compiler_params=pltpu.CompilerParams(
            collective_id=0, vmem_limit_bytes=100 * 1024 * 1024
        ),
    )(x, w_mat)


# device time: 309537 ns/iter; 2.5192x vs baseline; 1.0143x over previous
import jax
import jax.numpy as jnp
from jax import lax
from jax.experimental import pallas as pl
from jax.experimental.pallas import tpu as pltpu

N_DEV = 16
M = 4096
N = 2048
CHUNK = M // N_DEV
HALF = N // 2


def kernel(x, w_mat):
    def body(x_ref, w_ref, out_ref, comm_r, comm_l,
             send_r, recv_r, send_l, recv_l):
        my = lax.axis_index("i")
        left = lax.rem(my + N_DEV - 1, N_DEV)
        right = lax.rem(my + 1, N_DEV)

        barrier_sem = pltpu.get_barrier_semaphore()
        for nbr in (left, right):
            pl.semaphore_signal(
                barrier_sem, inc=1,
                device_id=(nbr,), device_id_type=pl.DeviceIdType.MESH,
            )
        pl.semaphore_wait(barrier_sem, 2)

        wr_b = w_ref[:, 0:HALF].astype(jnp.bfloat16)
        wl_b = w_ref[:, HALF:N].astype(jnp.bfloat16)

        def rows(c):
            return pl.ds(c * CHUNK, CHUNK)

        def partial(c, w_half):
            xc = x_ref[rows(c), :].astype(jnp.bfloat16)
            return jnp.dot(xc, w_half, preferred_element_type=jnp.float32)

        cols_r = pl.ds(0, HALF)
        cols_l = pl.ds(HALF, HALF)

        comm_r[0, :, :] = partial(my, wr_b).astype(jnp.bfloat16)
        comm_l[0, :, :] = partial(my, wl_b).astype(jnp.bfloat16)

        for h in range(2 * (N_DEV - 1)):
            s_slot = h % 2
            r_slot = (h + 1) % 2
            rdma_r = pltpu.make_async_remote_copy(
                src_ref=comm_r.at[s_slot],
                dst_ref=comm_r.at[r_slot],
                send_sem=send_r.at[s_slot],
                recv_sem=recv_r.at[r_slot],
                device_id=(right,),
                device_id_type=pl.DeviceIdType.MESH,
            )
            rdma_l = pltpu.make_async_remote_copy(
                src_ref=comm_l.at[s_slot],
                dst_ref=comm_l.at[r_slot],
                send_sem=send_l.at[s_slot],
                recv_sem=recv_l.at[r_slot],
                device_id=(left,),
                device_id_type=pl.DeviceIdType.MESH,
            )
            rdma_r.start()
            rdma_l.start()

            if h < N_DEV - 1:
                c_r = lax.rem(my + N_DEV - h - 1, N_DEV)
                c_l = lax.rem(my + h + 1, N_DEV)
                p_r = partial(c_r, wr_b)
                p_l = partial(c_l, wl_b)
            elif h >= N_DEV:
                t_prev = h - N_DEV
                cp_r = lax.rem(my + N_DEV - t_prev, N_DEV)
                cp_l = lax.rem(my + t_prev, N_DEV)
                out_ref[rows(cp_r), cols_r] = comm_r[s_slot, :, :].astype(
                    jnp.float32)
                out_ref[rows(cp_l), cols_l] = comm_l[s_slot, :, :].astype(
                    jnp.float32)

            rdma_r.wait()
            rdma_l.wait()

            if h < N_DEV - 1:
                summed = comm_r[r_slot, :, :].astype(jnp.float32) + p_r
                comm_r[r_slot, :, :] = summed.astype(jnp.bfloat16)
                if h == N_DEV - 2:
                    out_ref[rows(c_r), cols_r] = summed
                summed = comm_l[r_slot, :, :].astype(jnp.float32) + p_l
                comm_l[r_slot, :, :] = summed.astype(jnp.bfloat16)
                if h == N_DEV - 2:
                    out_ref[rows(c_l), cols_l] = summed

        t = N_DEV - 2
        c_r = lax.rem(my + N_DEV - t, N_DEV)
        c_l = lax.rem(my + t, N_DEV)
        last_r = (2 * (N_DEV - 1)) % 2
        out_ref[rows(c_r), cols_r] = comm_r[last_r, :, :].astype(jnp.float32)
        out_ref[rows(c_l), cols_l] = comm_l[last_r, :, :].astype(jnp.float32)

    comm_shape = (2, CHUNK, HALF)
    return pl.pallas_call(
        body,
        out_shape=jax.ShapeDtypeStruct((M, N), jnp.float32),
        in_specs=[
            pl.BlockSpec(memory_space=pltpu.VMEM),
            pl.BlockSpec(memory_space=pltpu.VMEM),
        ],
        out_specs=pl.BlockSpec(memory_space=pltpu.VMEM),
        scratch_shapes=[
            pltpu.VMEM(comm_shape, jnp.bfloat16),
            pltpu.VMEM(comm_shape, jnp.bfloat16),
            pltpu.SemaphoreType.DMA((2,)),
            pltpu.SemaphoreType.DMA((2,)),
            pltpu.SemaphoreType.DMA((2,)),
            pltpu.SemaphoreType.DMA((2,)),
        ],
        compiler_params=pltpu.CompilerParams(
            collective_id=0, vmem_limit_bytes=100 * 1024 * 1024
        ),
    )(x, w_mat)


# device time: 213666 ns/iter; 3.6496x vs baseline; 1.4487x over previous
import jax
import jax.numpy as jnp
from jax import lax
from jax.experimental import pallas as pl
from jax.experimental.pallas import tpu as pltpu

N_DEV = 16
M = 4096
N = 2048
CHUNK = M // N_DEV
HALF = N // 2
SUB = 2
SUBW = HALF // SUB
DEPTH = 4
HOPS = 2 * (N_DEV - 1)


def kernel(x, w_mat):
    def body(x_ref, w_ref, out_ref, *scratch):
        comm = {}
        send_sems = {}
        recv_sems = {}
        it = iter(scratch)
        for d in ("r", "l"):
            for s in range(SUB):
                comm[(d, s)] = next(it)
                send_sems[(d, s)] = next(it)
                recv_sems[(d, s)] = next(it)

        my = lax.axis_index("i")
        left = lax.rem(my + N_DEV - 1, N_DEV)
        right = lax.rem(my + 1, N_DEV)
        dir_target = {"r": right, "l": left}

        barrier_sem = pltpu.get_barrier_semaphore()
        for nbr in (left, right):
            pl.semaphore_signal(
                barrier_sem, inc=1,
                device_id=(nbr,), device_id_type=pl.DeviceIdType.MESH,
            )
        pl.semaphore_wait(barrier_sem, 2)

        w_b = {
            "r": w_ref[:, 0:HALF].astype(jnp.bfloat16),
            "l": w_ref[:, HALF:N].astype(jnp.bfloat16),
        }
        col_base = {"r": 0, "l": HALF}

        def rows(c):
            return pl.ds(c * CHUNK, CHUNK)

        def partial(c, d):
            xc = x_ref[rows(c), :].astype(jnp.bfloat16)
            return jnp.dot(xc, w_b[d], preferred_element_type=jnp.float32)

        def reduce_chunk(d, h):
            off = N_DEV - h - 1 if d == "r" else h + 1
            return lax.rem(my + off, N_DEV)

        def gather_chunk(d, t):
            off = N_DEV - t if d == "r" else t
            return lax.rem(my + off, N_DEV)

        def send_desc(d, s, h):
            return pltpu.make_async_remote_copy(
                src_ref=comm[(d, s)].at[h % DEPTH],
                dst_ref=comm[(d, s)].at[(h + 1) % DEPTH],
                send_sem=send_sems[(d, s)].at[h % DEPTH],
                recv_sem=recv_sems[(d, s)].at[(h + 1) % DEPTH],
                device_id=(dir_target[d],),
                device_id_type=pl.DeviceIdType.MESH,
            )

        sent = {}
        for d in ("r", "l"):
            p = partial(my, d).astype(jnp.bfloat16)
            for s in range(SUB):
                comm[(d, s)][0, :, :] = p[:, s * SUBW:(s + 1) * SUBW]
        for d in ("r", "l"):
            for s in range(SUB):
                rdma = send_desc(d, s, 0)
                rdma.start()
                sent[(d, s)] = [rdma]

        for h in range(HOPS):
            reduce_hop = h < N_DEV - 1
            p = {}
            if reduce_hop:
                for d in ("r", "l"):
                    p[d] = partial(reduce_chunk(d, h), d)

            for s in range(SUB):
                for d in ("r", "l"):
                    if h >= DEPTH - 1:
                        sent[(d, s)].pop(0).wait_send()
                    recv = send_desc(d, s, h)
                    recv.wait_recv()
                    slot = (h + 1) % DEPTH
                    cols = pl.ds(col_base[d] + s * SUBW, SUBW)
                    if reduce_hop:
                        summed = (
                            comm[(d, s)][slot, :, :].astype(jnp.float32)
                            + p[d][:, s * SUBW:(s + 1) * SUBW]
                        )
                        comm[(d, s)][slot, :, :] = summed.astype(jnp.bfloat16)
                        if h < HOPS - 1:
                            nxt = send_desc(d, s, h + 1)
                            nxt.start()
                            sent[(d, s)].append(nxt)
                        if h == N_DEV - 2:
                            out_ref[rows(reduce_chunk(d, h)), cols] = summed
                    else:
                        if h < HOPS - 1:
                            nxt = send_desc(d, s, h + 1)
                            nxt.start()
                            sent[(d, s)].append(nxt)
                        t = h - (N_DEV - 1)
                        c = gather_chunk(d, t)
                        out_ref[rows(c), cols] = comm[(d, s)][
                            slot, :, :].astype(jnp.float32)

        for d in ("r", "l"):
            for s in range(SUB):
                for rdma in sent[(d, s)]:
                    rdma.wait_send()

    scratch_shapes = []
    for _d in ("r", "l"):
        for _s in range(SUB):
            scratch_shapes.append(
                pltpu.VMEM((DEPTH, CHUNK, SUBW), jnp.bfloat16))
            scratch_shapes.append(pltpu.SemaphoreType.DMA((DEPTH,)))
            scratch_shapes.append(pltpu.SemaphoreType.DMA((DEPTH,)))

    return pl.pallas_call(
        body,
        out_shape=jax.ShapeDtypeStruct((M, N), jnp.float32),
        in_specs=[
            pl.BlockSpec(memory_space=pltpu.VMEM),
            pl.BlockSpec(memory_space=pltpu.VMEM),
        ],
        out_specs=pl.BlockSpec(memory_space=pltpu.VMEM),
        scratch_shapes=scratch_shapes,
        compiler_params=pltpu.CompilerParams(
            collective_id=0, vmem_limit_bytes=100 * 1024 * 1024
        ),
    )(x, w_mat)
